# baseline (speedup 1.0000x reference)
import functools
import math

import jax
import jax.numpy as jnp
from jax import lax
from jax.experimental import pallas as pl
from jax.experimental.pallas import tpu as pltpu
from jax.experimental.pallas import tpu_sc as plsc

_D = 32
_SCALE = math.sqrt(float(_D))
_NC = 2
_NS = 16
_NW = _NC * _NS
_C = 800
_NB = 4
_LA = 3
_NSEG = 2


@jax.jit
def _embed(tokens_flat, table):
    n = tokens_flat.shape[0] // _NSEG
    per_w = n // _NW
    n_chunks = per_w // _C
    n_groups = n_chunks // _NB

    mesh = plsc.VectorSubcoreMesh(core_axis_name="c", subcore_axis_name="s")

    def make_emb(seg):
        @functools.partial(
            pl.kernel,
            mesh=mesh,
            out_type=jax.ShapeDtypeStruct((n, _D), jnp.float32),
            scratch_types=[
                pltpu.VMEM((_NB, _C), jnp.int32),
                pltpu.VMEM((_NB, _C, _D), jnp.float32),
                pltpu.SemaphoreType.DMA((_NB,)),
                pltpu.SemaphoreType.DMA((_NB,)),
            ],
            compiler_params=pltpu.CompilerParams(use_tc_tiling_on_sc=False),
            name=f"emb_seg{seg}",
        )
        def emb(tok_hbm, tab_hbm, out_hbm, idx_v, rows_v, gsem, osem):
            wid = lax.axis_index("s") * _NC + lax.axis_index("c")
            base = wid * per_w
            tok0 = seg * n

            def start_gather(ci, b):
                off = base + ci * _C
                pltpu.sync_copy(
                    tok_hbm.at[pl.ds(tok0 + off, _C)], idx_v.at[b]
                )
                pltpu.make_async_copy(
                    tab_hbm.at[idx_v.at[b]], rows_v.at[b], gsem.at[b]
                ).start()

            for b in range(_LA):
                start_gather(b, b)

            def group(g, carry):
                ci0 = g * _NB
                for b in range(_NB):
                    ci = ci0 + b
                    off = base + ci * _C
                    pltpu.make_async_copy(
                        tab_hbm.at[idx_v.at[b]], rows_v.at[b], gsem.at[b]
                    ).wait()

                    @plsc.parallel_loop(0, _C, 1, unroll=8)
                    def _scale(i):
                        rows_v[b, i, pl.ds(0, 16)] = (
                            rows_v[b, i, pl.ds(0, 16)] * _SCALE
                        )
                        rows_v[b, i, pl.ds(16, 16)] = (
                            rows_v[b, i, pl.ds(16, 16)] * _SCALE
                        )

                    pltpu.make_async_copy(
                        rows_v.at[b], out_hbm.at[pl.ds(off, _C)], osem.at[b]
                    ).start()

                    bb = (b + _LA) % _NB

                    @pl.when(ci + _LA < n_chunks)
                    def _refill():
                        @pl.when(ci + _LA >= _NB)
                        def _drain_prev():
                            pltpu.make_async_copy(
                                rows_v.at[bb],
                                out_hbm.at[pl.ds(off, _C)],
                                osem.at[bb],
                            ).wait()

                        start_gather(ci + _LA, bb)

                return carry

            lax.fori_loop(0, n_groups, group, 0)

            for b in range(_NB):
                off = base + ((n_groups - 1) * _NB + b) * _C
                pltpu.make_async_copy(
                    rows_v.at[b], out_hbm.at[pl.ds(off, _C)], osem.at[b]
                ).wait()

        return emb

    outs = [make_emb(seg)(tokens_flat, table) for seg in range(_NSEG)]
    return jnp.concatenate(outs, axis=0)


def kernel(tokens, table):
    b, s = tokens.shape
    out = _embed(tokens.reshape(-1), table)
    return out.reshape(b, s, _D)


# speedup vs baseline: 4.5353x; 4.5353x over previous
import functools
import math

import jax
import jax.numpy as jnp
from jax import lax
from jax.experimental import pallas as pl
from jax.experimental.pallas import tpu as pltpu
from jax.experimental.pallas import tpu_sc as plsc

_D = 32
_SCALE = math.sqrt(float(_D))
_NC = 2
_NS = 16
_NW = _NC * _NS
_C = 800
_NB = 4
_LA = 3


@jax.jit
def _embed(tokens_flat, table):
    n = tokens_flat.shape[0]
    per_w = n // _NW
    n_chunks = per_w // _C
    n_groups = n_chunks // _NB

    mesh = plsc.VectorSubcoreMesh(core_axis_name="c", subcore_axis_name="s")

    @functools.partial(
        pl.kernel,
        mesh=mesh,
        out_type=jax.ShapeDtypeStruct((n, _D), jnp.float32),
        scratch_types=[
            pltpu.VMEM((_NB, _C), jnp.int32),
            pltpu.VMEM((_NB, _C, _D), jnp.float32),
            pltpu.SemaphoreType.DMA((_NB,)),
            pltpu.SemaphoreType.DMA((_NB,)),
        ],
        compiler_params=pltpu.CompilerParams(use_tc_tiling_on_sc=False),
    )
    def emb(tok_hbm, tab_hbm, out_hbm, idx_v, rows_v, gsem, osem):
        wid = lax.axis_index("s") * _NC + lax.axis_index("c")
        base = wid * per_w

        def start_gather(ci, b):
            off = base + ci * _C
            pltpu.sync_copy(tok_hbm.at[pl.ds(off, _C)], idx_v.at[b])
            pltpu.make_async_copy(
                tab_hbm.at[idx_v.at[b]], rows_v.at[b], gsem.at[b]
            ).start()

        for b in range(_LA):
            start_gather(b, b)

        def group(g, carry):
            ci0 = g * _NB
            for b in range(_NB):
                ci = ci0 + b
                off = base + ci * _C
                pltpu.make_async_copy(
                    tab_hbm.at[idx_v.at[b]], rows_v.at[b], gsem.at[b]
                ).wait()

                @plsc.parallel_loop(0, _C, 1, unroll=8)
                def _scale(i):
                    rows_v[b, i, pl.ds(0, 16)] = rows_v[b, i, pl.ds(0, 16)] * _SCALE
                    rows_v[b, i, pl.ds(16, 16)] = (
                        rows_v[b, i, pl.ds(16, 16)] * _SCALE
                    )

                pltpu.make_async_copy(
                    rows_v.at[b], out_hbm.at[pl.ds(off, _C)], osem.at[b]
                ).start()

                bb = (b + _LA) % _NB

                @pl.when(ci + _LA < n_chunks)
                def _refill():
                    @pl.when(ci + _LA >= _NB)
                    def _drain_prev():
                        pltpu.make_async_copy(
                            rows_v.at[bb],
                            out_hbm.at[pl.ds(off, _C)],
                            osem.at[bb],
                        ).wait()

                    start_gather(ci + _LA, bb)

            return carry

        lax.fori_loop(0, n_groups, group, 0)

        for b in range(_NB):
            off = base + ((n_groups - 1) * _NB + b) * _C
            pltpu.make_async_copy(
                rows_v.at[b], out_hbm.at[pl.ds(off, _C)], osem.at[b]
            ).wait()

    return emb(tokens_flat, table)


def kernel(tokens, table):
    b, s = tokens.shape
    out = _embed(tokens.reshape(-1), table)
    return out.reshape(b, s, _D)
